# baseline (speedup 1.0000x reference)
import functools

import jax
import jax.numpy as jnp
from jax.experimental import pallas as pl
from jax.experimental.pallas import tpu as pltpu

N, D, H, W, C = 1, 4, 224, 224, 96
N_WIN = 14
NUM_HEADS = 8
TOPK = 4
QK_DIM = C
DIM = C
SCALE = QK_DIM ** (-0.5)
R = 4

HW = H // N_WIN
WW = W // N_WIN
P2 = N_WIN * N_WIN
PIX = HW * WW
HD, WD = HW // R, WW // R
W2D = HD * WD
CPH = QK_DIM // NUM_HEADS
KW2 = TOPK * W2D
BD = NUM_HEADS * KW2


def _pool_matrix():
    import numpy as np
    P = np.zeros((W2D, PIX), dtype=np.float32)
    for pr in range(HD):
        for pc in range(WD):
            for dr in range(R):
                for dc in range(R):
                    P[pr * WD + pc, (pr * R + dr) * WW + (pc * R + dc)] = 1.0 / (R * R)
    return jnp.asarray(P)


def _dot(a, b, dims=(((1,), (0,)), ((), ())), prec=jax.lax.Precision.HIGHEST):
    return jax.lax.dot_general(a, b, dims, precision=prec,
                               preferred_element_type=jnp.float32)


_DEF = jax.lax.Precision.DEFAULT


def _k1_body(x_ref, wq_ref, wk_ref, wv_ref, bq_ref, bk_ref, bv_ref, P_ref,
             q_ref, kp_ref, vp_ref, qm_ref, km_ref):
    xv = x_ref[...].reshape(PIX, C)
    q = _dot(xv, wq_ref[...], prec=_DEF) + bq_ref[...]
    k = _dot(xv, wk_ref[...], prec=_DEF) + bk_ref[...]
    v = _dot(xv, wv_ref[...], prec=_DEF) + bv_ref[...]
    q_ref[...] = q.reshape(1, 1, PIX, QK_DIM)
    P = P_ref[...]
    kp_ref[...] = _dot(P, k).reshape(1, 1, W2D, QK_DIM)
    vp_ref[...] = _dot(P, v).reshape(1, 1, W2D, DIM)
    qm_ref[...] = jnp.mean(q, axis=0, keepdims=True).reshape(1, 1, 1, C)
    km_ref[...] = jnp.mean(k, axis=0, keepdims=True).reshape(1, 1, 1, C)


def _k2_body(qm_ref, km_ref, idx_ref):
    qw = qm_ref[...].reshape(P2, C)
    kw = km_ref[...].reshape(P2, C)
    lg = _dot(qw * SCALE, kw, (((1,), (1,)), ((), ())), prec=_DEF)
    cols = jax.lax.broadcasted_iota(jnp.int32, (P2, P2), 1)
    cur = lg
    picks = []
    for _ in range(TOPK):
        m = jnp.max(cur, axis=1, keepdims=True)
        eq = cur >= m
        am = jnp.min(jnp.where(eq, cols, jnp.int32(2 ** 30)), axis=1, keepdims=True)
        picks.append(am)
        cur = jnp.where(cols == am, -jnp.inf, cur)
    idx_ref[...] = jnp.concatenate(picks, axis=1).reshape(1, P2, TOPK)


def _k3_body(sref, q_ref, kp_ref, vp_ref, wo_ref, bo_ref, o_ref):
    t = pl.program_id(0)
    w = pl.program_id(1)
    q = q_ref[...].reshape(PIX, QK_DIM)
    krows = []
    vrows = []
    for kk in range(TOPK):
        i = sref[t, w, kk]
        krows.append(kp_ref[0, i])
        vrows.append(vp_ref[0, i])
    ks = jnp.concatenate(krows, axis=0)
    vs = jnp.concatenate(vrows, axis=0)
    rows = jax.lax.broadcasted_iota(jnp.int32, (BD, C), 0) // KW2
    colh = jax.lax.broadcasted_iota(jnp.int32, (BD, C), 1) // CPH
    hm = rows == colh
    Kb = jnp.where(hm, jnp.concatenate([ks] * NUM_HEADS, axis=0), 0.0)
    L = _dot(q * SCALE, Kb, (((1,), (1,)), ((), ())))
    parts = []
    for h in range(NUM_HEADS):
        Lh = L[:, h * KW2:(h + 1) * KW2]
        m = jnp.max(Lh, axis=1, keepdims=True)
        e = jnp.exp(Lh - m)
        parts.append(e / jnp.sum(e, axis=1, keepdims=True))
    A = jnp.concatenate(parts, axis=1)
    Vb = jnp.where(hm, jnp.concatenate([vs] * NUM_HEADS, axis=0), 0.0)
    out = _dot(A, Vb)
    o = _dot(out, wo_ref[...], prec=_DEF) + bo_ref[...]
    o_ref[...] = o.reshape(1, 1, HW, 1, WW, C)


@jax.jit
def kernel(x, w_qkv, b_qkv, w_o, b_o):
    xr = x.reshape(D, N_WIN, HW, N_WIN, WW, C)
    wq = w_qkv[:, :QK_DIM]
    wk = w_qkv[:, QK_DIM:2 * QK_DIM]
    wv = w_qkv[:, 2 * QK_DIM:]
    bq = b_qkv[:QK_DIM].reshape(1, QK_DIM)
    bk = b_qkv[QK_DIM:2 * QK_DIM].reshape(1, QK_DIM)
    bv = b_qkv[2 * QK_DIM:].reshape(1, DIM)
    bo = b_o.reshape(1, C)
    P = _pool_matrix()

    q_pix, kp, vp, qm, km = pl.pallas_call(
        _k1_body,
        grid=(D, P2),
        in_specs=[
            pl.BlockSpec((1, 1, HW, 1, WW, C),
                         lambda t, w: (t, w // N_WIN, 0, w % N_WIN, 0, 0)),
            pl.BlockSpec((C, QK_DIM), lambda t, w: (0, 0)),
            pl.BlockSpec((C, QK_DIM), lambda t, w: (0, 0)),
            pl.BlockSpec((C, DIM), lambda t, w: (0, 0)),
            pl.BlockSpec((1, QK_DIM), lambda t, w: (0, 0)),
            pl.BlockSpec((1, QK_DIM), lambda t, w: (0, 0)),
            pl.BlockSpec((1, DIM), lambda t, w: (0, 0)),
            pl.BlockSpec((W2D, PIX), lambda t, w: (0, 0)),
        ],
        out_specs=[
            pl.BlockSpec((1, 1, PIX, QK_DIM), lambda t, w: (t, w, 0, 0)),
            pl.BlockSpec((1, 1, W2D, QK_DIM), lambda t, w: (t, w, 0, 0)),
            pl.BlockSpec((1, 1, W2D, DIM), lambda t, w: (t, w, 0, 0)),
            pl.BlockSpec((1, 1, 1, C), lambda t, w: (t, w, 0, 0)),
            pl.BlockSpec((1, 1, 1, C), lambda t, w: (t, w, 0, 0)),
        ],
        out_shape=[
            jax.ShapeDtypeStruct((D, P2, PIX, QK_DIM), jnp.float32),
            jax.ShapeDtypeStruct((D, P2, W2D, QK_DIM), jnp.float32),
            jax.ShapeDtypeStruct((D, P2, W2D, DIM), jnp.float32),
            jax.ShapeDtypeStruct((D, P2, 1, C), jnp.float32),
            jax.ShapeDtypeStruct((D, P2, 1, C), jnp.float32),
        ],
    )(xr, wq, wk, wv, bq, bk, bv, P)

    qm3 = qm.reshape(D, P2, C)
    km3 = km.reshape(D, P2, C)
    r_idx = pl.pallas_call(
        _k2_body,
        grid=(D,),
        in_specs=[
            pl.BlockSpec((1, P2, C), lambda t: (t, 0, 0)),
            pl.BlockSpec((1, P2, C), lambda t: (jnp.minimum(t + 1, D - 1), 0, 0)),
        ],
        out_specs=pl.BlockSpec((1, P2, TOPK), lambda t: (t, 0, 0)),
        out_shape=jax.ShapeDtypeStruct((D, P2, TOPK), jnp.int32),
    )(qm3, km3)

    grid_spec = pltpu.PrefetchScalarGridSpec(
        num_scalar_prefetch=1,
        grid=(D, P2),
        in_specs=[
            pl.BlockSpec((1, 1, PIX, QK_DIM), lambda t, w, s: (t, w, 0, 0)),
            pl.BlockSpec((1, P2, W2D, QK_DIM),
                         lambda t, w, s: (jnp.minimum(t + 1, D - 1), 0, 0, 0)),
            pl.BlockSpec((1, P2, W2D, DIM),
                         lambda t, w, s: (jnp.minimum(t + 1, D - 1), 0, 0, 0)),
            pl.BlockSpec((C, C), lambda t, w, s: (0, 0)),
            pl.BlockSpec((1, C), lambda t, w, s: (0, 0)),
        ],
        out_specs=pl.BlockSpec((1, 1, HW, 1, WW, C),
                               lambda t, w, s: (t, w // N_WIN, 0, w % N_WIN, 0, 0)),
    )
    out = pl.pallas_call(
        _k3_body,
        grid_spec=grid_spec,
        out_shape=jax.ShapeDtypeStruct((D, N_WIN, HW, N_WIN, WW, C), jnp.float32),
    )(r_idx, q_pix, kp, vp, w_o, bo)

    return out.reshape(N, D, H, W, C)


# speedup vs baseline: 1.1058x; 1.1058x over previous
import functools

import jax
import jax.numpy as jnp
from jax.experimental import pallas as pl
from jax.experimental.pallas import tpu as pltpu

N, D, H, W, C = 1, 4, 224, 224, 96
N_WIN = 14
NUM_HEADS = 8
TOPK = 4
QK_DIM = C
DIM = C
SCALE = QK_DIM ** (-0.5)
R = 4

HW = H // N_WIN
WW = W // N_WIN
P2 = N_WIN * N_WIN
PIX = HW * WW
HD, WD = HW // R, WW // R
W2D = HD * WD
CPH = QK_DIM // NUM_HEADS
KW2 = TOPK * W2D
BD = NUM_HEADS * KW2


def _dot(a, b, dims=(((1,), (0,)), ((), ())), prec=jax.lax.Precision.HIGHEST):
    return jax.lax.dot_general(a, b, dims, precision=prec,
                               preferred_element_type=jnp.float32)


_DEF = jax.lax.Precision.DEFAULT


def _pool(a):
    cdim = a.shape[-1]
    colp = a.reshape(PIX // R, R, cdim).sum(axis=1)
    rowp = colp.reshape(HW // R, R, WD, cdim).sum(axis=1)
    return rowp.reshape(W2D, cdim) * (1.0 / (R * R))


def _k1_body(x_ref, wq_ref, wk_ref, wv_ref, bq_ref, bk_ref, bv_ref,
             kp_ref, vp_ref, qm_ref, km_ref):
    xv = x_ref[...].reshape(PIX, C)
    q = _dot(xv, wq_ref[...], prec=_DEF) + bq_ref[...]
    k = _dot(xv, wk_ref[...], prec=_DEF) + bk_ref[...]
    v = _dot(xv, wv_ref[...], prec=_DEF) + bv_ref[...]
    kp_ref[...] = _pool(k).reshape(1, 1, W2D, QK_DIM)
    vp_ref[...] = _pool(v).reshape(1, 1, W2D, DIM)
    qm_ref[...] = jnp.mean(q, axis=0, keepdims=True).reshape(1, 1, 1, C)
    km_ref[...] = jnp.mean(k, axis=0, keepdims=True).reshape(1, 1, 1, C)


def _k2_body(qm_ref, km_ref, idx_ref):
    qw = qm_ref[...].reshape(P2, C)
    kw = km_ref[...].reshape(P2, C)
    lg = _dot(qw * SCALE, kw, (((1,), (1,)), ((), ())), prec=_DEF)
    cols = jax.lax.broadcasted_iota(jnp.int32, (P2, P2), 1)
    cur = lg
    picks = []
    for _ in range(TOPK):
        m = jnp.max(cur, axis=1, keepdims=True)
        eq = cur >= m
        am = jnp.min(jnp.where(eq, cols, jnp.int32(2 ** 30)), axis=1, keepdims=True)
        picks.append(am)
        cur = jnp.where(cols == am, -jnp.inf, cur)
    idx_ref[...] = jnp.concatenate(picks, axis=1).reshape(1, P2, TOPK)


def _k3_body(sref, x_ref, wq_ref, bq_ref, kp_ref, vp_ref, wo_ref, bo_ref, o_ref):
    t = pl.program_id(0)
    w = pl.program_id(1)
    xv = x_ref[...].reshape(PIX, C)
    q = _dot(xv, wq_ref[...], prec=_DEF) + bq_ref[...]
    krows = []
    vrows = []
    for kk in range(TOPK):
        i = sref[t, w, kk]
        krows.append(kp_ref[0, i])
        vrows.append(vp_ref[0, i])
    ks = jnp.concatenate(krows, axis=0)
    vs = jnp.concatenate(vrows, axis=0)
    rows = jax.lax.broadcasted_iota(jnp.int32, (BD, C), 0) // KW2
    colh = jax.lax.broadcasted_iota(jnp.int32, (BD, C), 1) // CPH
    hm = rows == colh
    Kb = jnp.where(hm, jnp.concatenate([ks] * NUM_HEADS, axis=0), 0.0)
    L = _dot(q * SCALE, Kb, (((1,), (1,)), ((), ())), prec=_DEF)
    parts = []
    for h in range(NUM_HEADS):
        Lh = L[:, h * KW2:(h + 1) * KW2]
        m = jnp.max(Lh, axis=1, keepdims=True)
        e = jnp.exp(Lh - m)
        parts.append(e / jnp.sum(e, axis=1, keepdims=True))
    A = jnp.concatenate(parts, axis=1)
    Vb = jnp.where(hm, jnp.concatenate([vs] * NUM_HEADS, axis=0), 0.0)
    out = _dot(A, Vb, prec=_DEF)
    o = _dot(out, wo_ref[...], prec=_DEF) + bo_ref[...]
    o_ref[...] = o.reshape(1, 1, HW, 1, WW, C)


@jax.jit
def kernel(x, w_qkv, b_qkv, w_o, b_o):
    xr = x.reshape(D, N_WIN, HW, N_WIN, WW, C)
    wq = w_qkv[:, :QK_DIM]
    wk = w_qkv[:, QK_DIM:2 * QK_DIM]
    wv = w_qkv[:, 2 * QK_DIM:]
    bq = b_qkv[:QK_DIM].reshape(1, QK_DIM)
    bk = b_qkv[QK_DIM:2 * QK_DIM].reshape(1, QK_DIM)
    bv = b_qkv[2 * QK_DIM:].reshape(1, DIM)
    bo = b_o.reshape(1, C)

    kp, vp, qm, km = pl.pallas_call(
        _k1_body,
        grid=(D, P2),
        in_specs=[
            pl.BlockSpec((1, 1, HW, 1, WW, C),
                         lambda t, w: (t, w // N_WIN, 0, w % N_WIN, 0, 0)),
            pl.BlockSpec((C, QK_DIM), lambda t, w: (0, 0)),
            pl.BlockSpec((C, QK_DIM), lambda t, w: (0, 0)),
            pl.BlockSpec((C, DIM), lambda t, w: (0, 0)),
            pl.BlockSpec((1, QK_DIM), lambda t, w: (0, 0)),
            pl.BlockSpec((1, QK_DIM), lambda t, w: (0, 0)),
            pl.BlockSpec((1, DIM), lambda t, w: (0, 0)),
        ],
        out_specs=[
            pl.BlockSpec((1, 1, W2D, QK_DIM), lambda t, w: (t, w, 0, 0)),
            pl.BlockSpec((1, 1, W2D, DIM), lambda t, w: (t, w, 0, 0)),
            pl.BlockSpec((1, 1, 1, C), lambda t, w: (t, w, 0, 0)),
            pl.BlockSpec((1, 1, 1, C), lambda t, w: (t, w, 0, 0)),
        ],
        out_shape=[
            jax.ShapeDtypeStruct((D, P2, W2D, QK_DIM), jnp.float32),
            jax.ShapeDtypeStruct((D, P2, W2D, DIM), jnp.float32),
            jax.ShapeDtypeStruct((D, P2, 1, C), jnp.float32),
            jax.ShapeDtypeStruct((D, P2, 1, C), jnp.float32),
        ],
    )(xr, wq, wk, wv, bq, bk, bv)

    qm3 = qm.reshape(D, P2, C)
    km3 = km.reshape(D, P2, C)
    r_idx = pl.pallas_call(
        _k2_body,
        grid=(D,),
        in_specs=[
            pl.BlockSpec((1, P2, C), lambda t: (t, 0, 0)),
            pl.BlockSpec((1, P2, C), lambda t: (jnp.minimum(t + 1, D - 1), 0, 0)),
        ],
        out_specs=pl.BlockSpec((1, P2, TOPK), lambda t: (t, 0, 0)),
        out_shape=jax.ShapeDtypeStruct((D, P2, TOPK), jnp.int32),
    )(qm3, km3)

    grid_spec = pltpu.PrefetchScalarGridSpec(
        num_scalar_prefetch=1,
        grid=(D, P2),
        in_specs=[
            pl.BlockSpec((1, 1, HW, 1, WW, C),
                         lambda t, w, s: (t, w // N_WIN, 0, w % N_WIN, 0, 0)),
            pl.BlockSpec((C, QK_DIM), lambda t, w, s: (0, 0)),
            pl.BlockSpec((1, QK_DIM), lambda t, w, s: (0, 0)),
            pl.BlockSpec((1, P2, W2D, QK_DIM),
                         lambda t, w, s: (jnp.minimum(t + 1, D - 1), 0, 0, 0)),
            pl.BlockSpec((1, P2, W2D, DIM),
                         lambda t, w, s: (jnp.minimum(t + 1, D - 1), 0, 0, 0)),
            pl.BlockSpec((C, C), lambda t, w, s: (0, 0)),
            pl.BlockSpec((1, C), lambda t, w, s: (0, 0)),
        ],
        out_specs=pl.BlockSpec((1, 1, HW, 1, WW, C),
                               lambda t, w, s: (t, w // N_WIN, 0, w % N_WIN, 0, 0)),
    )
    out = pl.pallas_call(
        _k3_body,
        grid_spec=grid_spec,
        out_shape=jax.ShapeDtypeStruct((D, N_WIN, HW, N_WIN, WW, C), jnp.float32),
    )(r_idx, xr, wq, bq, kp, vp, w_o, bo)

    return out.reshape(N, D, H, W, C)


# speedup vs baseline: 1.9471x; 1.7609x over previous
import functools

import jax
import jax.numpy as jnp
from jax.experimental import pallas as pl
from jax.experimental.pallas import tpu as pltpu

N, D, H, W, C = 1, 4, 224, 224, 96
N_WIN = 14
NUM_HEADS = 8
TOPK = 4
QK_DIM = C
DIM = C
SCALE = QK_DIM ** (-0.5)
R = 4

HW = H // N_WIN
WW = W // N_WIN
P2 = N_WIN * N_WIN
PIX = HW * WW
HD, WD = HW // R, WW // R
W2D = HD * WD
CPH = QK_DIM // NUM_HEADS
KW2 = TOPK * W2D
BD = NUM_HEADS * KW2


def _dot(a, b, dims=(((1,), (0,)), ((), ())), prec=jax.lax.Precision.HIGHEST):
    return jax.lax.dot_general(a, b, dims, precision=prec,
                               preferred_element_type=jnp.float32)


_DEF = jax.lax.Precision.DEFAULT


def _pool(a):
    cdim = a.shape[-1]
    colp = a.reshape(PIX // R, R, cdim).sum(axis=1)
    rowp = colp.reshape(HW // R, R, WD, cdim).sum(axis=1)
    return rowp.reshape(W2D, cdim) * (1.0 / (R * R))


def _k1_body(x_ref, wq_ref, wk_ref, wv_ref, bq_ref, bk_ref, bv_ref,
             kp_ref, vp_ref, qm_ref, km_ref):
    xv = x_ref[...].reshape(PIX, C)
    q = _dot(xv, wq_ref[...], prec=_DEF) + bq_ref[...]
    k = _dot(xv, wk_ref[...], prec=_DEF) + bk_ref[...]
    v = _dot(xv, wv_ref[...], prec=_DEF) + bv_ref[...]
    kp_ref[...] = _pool(k).reshape(1, 1, W2D, QK_DIM)
    vp_ref[...] = _pool(v).reshape(1, 1, W2D, DIM)
    qm_ref[...] = jnp.mean(q, axis=0, keepdims=True).reshape(1, 1, 1, C)
    km_ref[...] = jnp.mean(k, axis=0, keepdims=True).reshape(1, 1, 1, C)


def _k2_body(qm_ref, km_ref, idx_ref):
    qw = qm_ref[...].reshape(P2, C)
    kw = km_ref[...].reshape(P2, C)
    lg = _dot(qw * SCALE, kw, (((1,), (1,)), ((), ())), prec=_DEF)
    cols = jax.lax.broadcasted_iota(jnp.int32, (P2, P2), 1)
    cur = lg
    picks = []
    for _ in range(TOPK):
        m = jnp.max(cur, axis=1, keepdims=True)
        eq = cur >= m
        am = jnp.min(jnp.where(eq, cols, jnp.int32(2 ** 30)), axis=1, keepdims=True)
        picks.append(am)
        cur = jnp.where(cols == am, -jnp.inf, cur)
    idx_ref[...] = jnp.concatenate(picks, axis=1).reshape(1, P2, TOPK)


def _k3_body(sref, x_ref, wq_ref, bq_ref, kp_ref, vp_ref, wo_ref, bo_ref, o_ref):
    t = pl.program_id(0)
    w = pl.program_id(1)
    xv = x_ref[...].reshape(PIX, C)
    q = _dot(xv, wq_ref[...], prec=_DEF) + bq_ref[...]
    krows = []
    vrows = []
    for kk in range(TOPK):
        i = sref[t, w, kk]
        krows.append(kp_ref[0, i])
        vrows.append(vp_ref[0, i])
    ks = jnp.concatenate(krows, axis=0)
    vs = jnp.concatenate(vrows, axis=0)
    rows = jax.lax.broadcasted_iota(jnp.int32, (BD, C), 0) // KW2
    colh = jax.lax.broadcasted_iota(jnp.int32, (BD, C), 1) // CPH
    hm = rows == colh
    Kb = jnp.where(hm, jnp.concatenate([ks] * NUM_HEADS, axis=0), 0.0)
    L = _dot(q * SCALE, Kb, (((1,), (1,)), ((), ())), prec=_DEF)
    m = jnp.max(L, axis=1, keepdims=True)
    e = jnp.exp(L - m)
    eb = jnp.where(
        jax.lax.broadcasted_iota(jnp.int32, (BD, NUM_HEADS), 0) // KW2
        == jax.lax.broadcasted_iota(jnp.int32, (BD, NUM_HEADS), 1), 1.0, 0.0)
    s = _dot(e, eb, prec=_DEF)
    rec = 1.0 / jnp.maximum(s, 1e-30)
    Vb = jnp.where(hm, jnp.concatenate([vs] * NUM_HEADS, axis=0), 0.0)
    ou = _dot(e, Vb, prec=_DEF)
    ex = jnp.where(
        jax.lax.broadcasted_iota(jnp.int32, (NUM_HEADS, C), 0)
        == jax.lax.broadcasted_iota(jnp.int32, (NUM_HEADS, C), 1) // CPH, 1.0, 0.0)
    out = ou * _dot(rec, ex, prec=_DEF)
    o = _dot(out, wo_ref[...], prec=_DEF) + bo_ref[...]
    o_ref[...] = o.reshape(1, HW, WW, C)


@jax.jit
def kernel(x, w_qkv, b_qkv, w_o, b_o):
    xr = x.reshape(D, H, W, C)
    wq = w_qkv[:, :QK_DIM]
    wk = w_qkv[:, QK_DIM:2 * QK_DIM]
    wv = w_qkv[:, 2 * QK_DIM:]
    bq = b_qkv[:QK_DIM].reshape(1, QK_DIM)
    bk = b_qkv[QK_DIM:2 * QK_DIM].reshape(1, QK_DIM)
    bv = b_qkv[2 * QK_DIM:].reshape(1, DIM)
    bo = b_o.reshape(1, C)

    kp, vp, qm, km = pl.pallas_call(
        _k1_body,
        grid=(D, P2),
        in_specs=[
            pl.BlockSpec((1, HW, WW, C),
                         lambda t, w: (t, w // N_WIN, w % N_WIN, 0)),
            pl.BlockSpec((C, QK_DIM), lambda t, w: (0, 0)),
            pl.BlockSpec((C, QK_DIM), lambda t, w: (0, 0)),
            pl.BlockSpec((C, DIM), lambda t, w: (0, 0)),
            pl.BlockSpec((1, QK_DIM), lambda t, w: (0, 0)),
            pl.BlockSpec((1, QK_DIM), lambda t, w: (0, 0)),
            pl.BlockSpec((1, DIM), lambda t, w: (0, 0)),
        ],
        out_specs=[
            pl.BlockSpec((1, 1, W2D, QK_DIM), lambda t, w: (t, w, 0, 0)),
            pl.BlockSpec((1, 1, W2D, DIM), lambda t, w: (t, w, 0, 0)),
            pl.BlockSpec((1, 1, 1, C), lambda t, w: (t, w, 0, 0)),
            pl.BlockSpec((1, 1, 1, C), lambda t, w: (t, w, 0, 0)),
        ],
        out_shape=[
            jax.ShapeDtypeStruct((D, P2, W2D, QK_DIM), jnp.float32),
            jax.ShapeDtypeStruct((D, P2, W2D, DIM), jnp.float32),
            jax.ShapeDtypeStruct((D, P2, 1, C), jnp.float32),
            jax.ShapeDtypeStruct((D, P2, 1, C), jnp.float32),
        ],
    )(xr, wq, wk, wv, bq, bk, bv)

    qm3 = qm.reshape(D, P2, C)
    km3 = km.reshape(D, P2, C)
    r_idx = pl.pallas_call(
        _k2_body,
        grid=(D,),
        in_specs=[
            pl.BlockSpec((1, P2, C), lambda t: (t, 0, 0)),
            pl.BlockSpec((1, P2, C), lambda t: (jnp.minimum(t + 1, D - 1), 0, 0)),
        ],
        out_specs=pl.BlockSpec((1, P2, TOPK), lambda t: (t, 0, 0)),
        out_shape=jax.ShapeDtypeStruct((D, P2, TOPK), jnp.int32),
    )(qm3, km3)

    grid_spec = pltpu.PrefetchScalarGridSpec(
        num_scalar_prefetch=1,
        grid=(D, P2),
        in_specs=[
            pl.BlockSpec((1, HW, WW, C),
                         lambda t, w, s: (t, w // N_WIN, w % N_WIN, 0)),
            pl.BlockSpec((C, QK_DIM), lambda t, w, s: (0, 0)),
            pl.BlockSpec((1, QK_DIM), lambda t, w, s: (0, 0)),
            pl.BlockSpec((1, P2, W2D, QK_DIM),
                         lambda t, w, s: (jnp.minimum(t + 1, D - 1), 0, 0, 0)),
            pl.BlockSpec((1, P2, W2D, DIM),
                         lambda t, w, s: (jnp.minimum(t + 1, D - 1), 0, 0, 0)),
            pl.BlockSpec((C, C), lambda t, w, s: (0, 0)),
            pl.BlockSpec((1, C), lambda t, w, s: (0, 0)),
        ],
        out_specs=pl.BlockSpec((1, HW, WW, C),
                               lambda t, w, s: (t, w // N_WIN, w % N_WIN, 0)),
    )
    out = pl.pallas_call(
        _k3_body,
        grid_spec=grid_spec,
        out_shape=jax.ShapeDtypeStruct((D, H, W, C), jnp.float32),
    )(r_idx, xr, wq, bq, kp, vp, w_o, bo)

    return out.reshape(N, D, H, W, C)


# speedup vs baseline: 1.9666x; 1.0100x over previous
import functools

import jax
import jax.numpy as jnp
from jax.experimental import pallas as pl
from jax.experimental.pallas import tpu as pltpu

N, D, H, W, C = 1, 4, 224, 224, 96
N_WIN = 14
NUM_HEADS = 8
TOPK = 4
QK_DIM = C
DIM = C
SCALE = QK_DIM ** (-0.5)
R = 4

HW = H // N_WIN
WW = W // N_WIN
P2 = N_WIN * N_WIN
PIX = HW * WW
HD, WD = HW // R, WW // R
W2D = HD * WD
CPH = QK_DIM // NUM_HEADS
KW2 = TOPK * W2D
BD = NUM_HEADS * KW2


def _dot(a, b, dims=(((1,), (0,)), ((), ())), prec=jax.lax.Precision.HIGHEST):
    return jax.lax.dot_general(a, b, dims, precision=prec,
                               preferred_element_type=jnp.float32)


_DEF = jax.lax.Precision.DEFAULT


def _pool(a):
    cdim = a.shape[-1]
    colp = a.reshape(PIX // R, R, cdim).sum(axis=1)
    rowp = colp.reshape(HW // R, R, WD, cdim).sum(axis=1)
    return rowp.reshape(W2D, cdim) * (1.0 / (R * R))


def _k1_body(x_ref, wq_ref, wk_ref, wv_ref, bq_ref, bk_ref, bv_ref,
             q_ref, kp_ref, vp_ref, qm_ref, km_ref):
    xv = x_ref[...].reshape(PIX, C)
    q = _dot(xv, wq_ref[...], prec=_DEF) + bq_ref[...]
    k = _dot(xv, wk_ref[...], prec=_DEF) + bk_ref[...]
    v = _dot(xv, wv_ref[...], prec=_DEF) + bv_ref[...]
    q_ref[...] = q.reshape(1, 1, PIX, C)
    kp_ref[...] = _pool(k).reshape(1, 1, W2D, QK_DIM)
    vp_ref[...] = _pool(v).reshape(1, 1, W2D, DIM)
    qm_ref[...] = jnp.mean(q, axis=0, keepdims=True).reshape(1, 1, 1, C)
    km_ref[...] = jnp.mean(k, axis=0, keepdims=True).reshape(1, 1, 1, C)


def _k2_body(qm_ref, km_ref, idx_ref):
    qw = qm_ref[...].reshape(P2, C)
    kw = km_ref[...].reshape(P2, C)
    lg = _dot(qw * SCALE, kw, (((1,), (1,)), ((), ())), prec=_DEF)
    cols = jax.lax.broadcasted_iota(jnp.int32, (P2, P2), 1)
    cur = lg
    picks = []
    for _ in range(TOPK):
        m = jnp.max(cur, axis=1, keepdims=True)
        eq = cur >= m
        am = jnp.min(jnp.where(eq, cols, jnp.int32(2 ** 30)), axis=1, keepdims=True)
        picks.append(am)
        cur = jnp.where(cols == am, -jnp.inf, cur)
    idx_ref[...] = jnp.concatenate(picks, axis=1).reshape(1, P2, TOPK)


def _k3_body(sref, q_ref, kp_ref, vp_ref, wo_ref, bo_ref, o_ref):
    t = pl.program_id(0)
    w = pl.program_id(1)
    q = q_ref[...].reshape(PIX, C)
    krows = []
    vrows = []
    for kk in range(TOPK):
        i = sref[t, w, kk]
        krows.append(kp_ref[0, i])
        vrows.append(vp_ref[0, i])
    ks = jnp.concatenate(krows, axis=0)
    vs = jnp.concatenate(vrows, axis=0)
    rows = jax.lax.broadcasted_iota(jnp.int32, (BD, C), 0) // KW2
    colh = jax.lax.broadcasted_iota(jnp.int32, (BD, C), 1) // CPH
    hm = rows == colh
    Kb = jnp.where(hm, jnp.concatenate([ks] * NUM_HEADS, axis=0), 0.0)
    L = _dot(q * SCALE, Kb, (((1,), (1,)), ((), ())), prec=_DEF)
    m = jnp.max(L, axis=1, keepdims=True)
    e = jnp.exp(L - m)
    eb = jnp.where(
        jax.lax.broadcasted_iota(jnp.int32, (BD, NUM_HEADS), 0) // KW2
        == jax.lax.broadcasted_iota(jnp.int32, (BD, NUM_HEADS), 1), 1.0, 0.0)
    s = _dot(e, eb, prec=_DEF)
    rec = 1.0 / jnp.maximum(s, 1e-30)
    Vb = jnp.where(hm, jnp.concatenate([vs] * NUM_HEADS, axis=0), 0.0)
    ou = _dot(e, Vb, prec=_DEF)
    ex = jnp.where(
        jax.lax.broadcasted_iota(jnp.int32, (NUM_HEADS, C), 0)
        == jax.lax.broadcasted_iota(jnp.int32, (NUM_HEADS, C), 1) // CPH, 1.0, 0.0)
    out = ou * _dot(rec, ex, prec=_DEF)
    o = _dot(out, wo_ref[...], prec=_DEF) + bo_ref[...]
    o_ref[...] = o.reshape(1, HW, WW, C)


@jax.jit
def kernel(x, w_qkv, b_qkv, w_o, b_o):
    xr = x.reshape(D, H, W, C)
    wq = w_qkv[:, :QK_DIM]
    wk = w_qkv[:, QK_DIM:2 * QK_DIM]
    wv = w_qkv[:, 2 * QK_DIM:]
    bq = b_qkv[:QK_DIM].reshape(1, QK_DIM)
    bk = b_qkv[QK_DIM:2 * QK_DIM].reshape(1, QK_DIM)
    bv = b_qkv[2 * QK_DIM:].reshape(1, DIM)
    bo = b_o.reshape(1, C)

    q_pix, kp, vp, qm, km = pl.pallas_call(
        _k1_body,
        grid=(D, P2),
        in_specs=[
            pl.BlockSpec((1, HW, WW, C),
                         lambda t, w: (t, w // N_WIN, w % N_WIN, 0)),
            pl.BlockSpec((C, QK_DIM), lambda t, w: (0, 0)),
            pl.BlockSpec((C, QK_DIM), lambda t, w: (0, 0)),
            pl.BlockSpec((C, DIM), lambda t, w: (0, 0)),
            pl.BlockSpec((1, QK_DIM), lambda t, w: (0, 0)),
            pl.BlockSpec((1, QK_DIM), lambda t, w: (0, 0)),
            pl.BlockSpec((1, DIM), lambda t, w: (0, 0)),
        ],
        out_specs=[
            pl.BlockSpec((1, 1, PIX, C), lambda t, w: (t, w, 0, 0)),
            pl.BlockSpec((1, 1, W2D, QK_DIM), lambda t, w: (t, w, 0, 0)),
            pl.BlockSpec((1, 1, W2D, DIM), lambda t, w: (t, w, 0, 0)),
            pl.BlockSpec((1, 1, 1, C), lambda t, w: (t, w, 0, 0)),
            pl.BlockSpec((1, 1, 1, C), lambda t, w: (t, w, 0, 0)),
        ],
        out_shape=[
            jax.ShapeDtypeStruct((D, P2, PIX, C), jnp.float32),
            jax.ShapeDtypeStruct((D, P2, W2D, QK_DIM), jnp.float32),
            jax.ShapeDtypeStruct((D, P2, W2D, DIM), jnp.float32),
            jax.ShapeDtypeStruct((D, P2, 1, C), jnp.float32),
            jax.ShapeDtypeStruct((D, P2, 1, C), jnp.float32),
        ],
    )(xr, wq, wk, wv, bq, bk, bv)

    r_idx = pl.pallas_call(
        _k2_body,
        grid=(D,),
        in_specs=[
            pl.BlockSpec((1, P2, 1, C), lambda t: (t, 0, 0, 0)),
            pl.BlockSpec((1, P2, 1, C),
                         lambda t: (jnp.minimum(t + 1, D - 1), 0, 0, 0)),
        ],
        out_specs=pl.BlockSpec((1, P2, TOPK), lambda t: (t, 0, 0)),
        out_shape=jax.ShapeDtypeStruct((D, P2, TOPK), jnp.int32),
    )(qm, km)

    grid_spec = pltpu.PrefetchScalarGridSpec(
        num_scalar_prefetch=1,
        grid=(D, P2),
        in_specs=[
            pl.BlockSpec((1, 1, PIX, C), lambda t, w, s: (t, w, 0, 0)),
            pl.BlockSpec((1, P2, W2D, QK_DIM),
                         lambda t, w, s: (jnp.minimum(t + 1, D - 1), 0, 0, 0)),
            pl.BlockSpec((1, P2, W2D, DIM),
                         lambda t, w, s: (jnp.minimum(t + 1, D - 1), 0, 0, 0)),
            pl.BlockSpec((C, C), lambda t, w, s: (0, 0)),
            pl.BlockSpec((1, C), lambda t, w, s: (0, 0)),
        ],
        out_specs=pl.BlockSpec((1, HW, WW, C),
                               lambda t, w, s: (t, w // N_WIN, w % N_WIN, 0)),
    )
    out = pl.pallas_call(
        _k3_body,
        grid_spec=grid_spec,
        out_shape=jax.ShapeDtypeStruct((D, H, W, C), jnp.float32),
    )(r_idx, q_pix, kp, vp, w_o, bo)

    return out.reshape(N, D, H, W, C)


# speedup vs baseline: 3.4898x; 1.7746x over previous
import functools

import jax
import jax.numpy as jnp
from jax.experimental import pallas as pl
from jax.experimental.pallas import tpu as pltpu

N, D, H, W, C = 1, 4, 224, 224, 96
N_WIN = 14
NUM_HEADS = 8
TOPK = 4
QK_DIM = C
DIM = C
SCALE = QK_DIM ** (-0.5)
R = 4

HW = H // N_WIN
WW = W // N_WIN
P2 = N_WIN * N_WIN
PIX = HW * WW
HD, WD = HW // R, WW // R
W2D = HD * WD
CPH = QK_DIM // NUM_HEADS
KW2 = TOPK * W2D
BD = NUM_HEADS * KW2


def _dot(a, b, dims=(((1,), (0,)), ((), ())), prec=jax.lax.Precision.HIGHEST):
    return jax.lax.dot_general(a, b, dims, precision=prec,
                               preferred_element_type=jnp.float32)


_DEF = jax.lax.Precision.DEFAULT


def _k1_body(x_ref, wq_ref, wk_ref, wv_ref, bq_ref, bk_ref, bv_ref,
             q_ref, kp_ref, vp_ref, qm_ref, km_ref):
    xb = x_ref[...]
    qs = []
    ks = []
    vs = []
    for r in range(HW):
        x_r = xb[0, r]
        qs.append(_dot(x_r, wq_ref[...], (((0,), (0,)), ((), ())), prec=_DEF)
                  + bq_ref[...])
        ks.append(_dot(x_r, wk_ref[...], (((0,), (0,)), ((), ())), prec=_DEF)
                  + bk_ref[...])
        vs.append(_dot(x_r, wv_ref[...], (((0,), (0,)), ((), ())), prec=_DEF)
                  + bv_ref[...])
    Q = jnp.stack(qs, axis=0)
    qw = Q.reshape(HW, N_WIN, WW, C).transpose(1, 0, 2, 3).reshape(1, N_WIN, PIX, C)
    q_ref[...] = qw
    pc_op = jnp.where(
        jax.lax.broadcasted_iota(jnp.int32, (W // R, W), 0)
        == jax.lax.broadcasted_iota(jnp.int32, (W // R, W), 1) // R, 1.0, 0.0)
    for dst, srcs in ((kp_ref, ks), (vp_ref, vs)):
        pooled = []
        for pr in range(HD):
            rsumT = srcs[R * pr]
            for dr in range(1, R):
                rsumT = rsumT + srcs[R * pr + dr]
            pooled.append(_dot(pc_op, rsumT))
        pool = jnp.stack(pooled, axis=0) * (1.0 / (R * R))
        pool = pool.reshape(HD, N_WIN, WD, C).transpose(1, 0, 2, 3)
        dst[...] = pool.reshape(1, N_WIN, W2D, C)
    m_op = jnp.where(
        jax.lax.broadcasted_iota(jnp.int32, (N_WIN, W), 0)
        == jax.lax.broadcasted_iota(jnp.int32, (N_WIN, W), 1) // WW, 1.0, 0.0)
    sq = qs[0]
    sk = ks[0]
    for r in range(1, HW):
        sq = sq + qs[r]
        sk = sk + ks[r]
    qm_ref[...] = (_dot(m_op, sq) * (1.0 / PIX)).reshape(1, N_WIN, 1, C)
    km_ref[...] = (_dot(m_op, sk) * (1.0 / PIX)).reshape(1, N_WIN, 1, C)


def _k2_body(qm_ref, km_ref, idx_ref):
    qw = qm_ref[...].reshape(P2, C)
    kw = km_ref[...].reshape(P2, C)
    lg = _dot(qw * SCALE, kw, (((1,), (1,)), ((), ())), prec=_DEF)
    cols = jax.lax.broadcasted_iota(jnp.int32, (P2, P2), 1)
    cur = lg
    picks = []
    for _ in range(TOPK):
        m = jnp.max(cur, axis=1, keepdims=True)
        eq = cur >= m
        am = jnp.min(jnp.where(eq, cols, jnp.int32(2 ** 30)), axis=1, keepdims=True)
        picks.append(am)
        cur = jnp.where(cols == am, -jnp.inf, cur)
    idx_ref[...] = jnp.concatenate(picks, axis=1).reshape(1, P2, TOPK)


def _k3_body(sref, q_ref, kp_ref, vp_ref, wo_ref, bo_ref, o_ref):
    t = pl.program_id(0)
    w = pl.program_id(1)
    q = q_ref[...].reshape(PIX, C)
    krows = []
    vrows = []
    for kk in range(TOPK):
        i = sref[t, w, kk]
        krows.append(kp_ref[0, i])
        vrows.append(vp_ref[0, i])
    ks = jnp.concatenate(krows, axis=0)
    vs = jnp.concatenate(vrows, axis=0)
    rows = jax.lax.broadcasted_iota(jnp.int32, (BD, C), 0) // KW2
    colh = jax.lax.broadcasted_iota(jnp.int32, (BD, C), 1) // CPH
    hm = rows == colh
    Kb = jnp.where(hm, jnp.concatenate([ks] * NUM_HEADS, axis=0), 0.0)
    L = _dot(q * SCALE, Kb, (((1,), (1,)), ((), ())), prec=_DEF)
    m = jnp.max(L, axis=1, keepdims=True)
    e = jnp.exp(L - m)
    eb = jnp.where(
        jax.lax.broadcasted_iota(jnp.int32, (BD, NUM_HEADS), 0) // KW2
        == jax.lax.broadcasted_iota(jnp.int32, (BD, NUM_HEADS), 1), 1.0, 0.0)
    s = _dot(e, eb, prec=_DEF)
    rec = 1.0 / jnp.maximum(s, 1e-30)
    Vb = jnp.where(hm, jnp.concatenate([vs] * NUM_HEADS, axis=0), 0.0)
    ou = _dot(e, Vb, prec=_DEF)
    ex = jnp.where(
        jax.lax.broadcasted_iota(jnp.int32, (NUM_HEADS, C), 0)
        == jax.lax.broadcasted_iota(jnp.int32, (NUM_HEADS, C), 1) // CPH, 1.0, 0.0)
    out = ou * _dot(rec, ex, prec=_DEF)
    o = _dot(out, wo_ref[...], prec=_DEF) + bo_ref[...]
    o_ref[...] = o.reshape(1, HW, WW, C)


@jax.jit
def kernel(x, w_qkv, b_qkv, w_o, b_o):
    xt = jnp.transpose(x.reshape(D, H, W, C), (0, 1, 3, 2))
    wq = w_qkv[:, :QK_DIM]
    wk = w_qkv[:, QK_DIM:2 * QK_DIM]
    wv = w_qkv[:, 2 * QK_DIM:]
    bq = b_qkv[:QK_DIM].reshape(1, QK_DIM)
    bk = b_qkv[QK_DIM:2 * QK_DIM].reshape(1, QK_DIM)
    bv = b_qkv[2 * QK_DIM:].reshape(1, DIM)
    bo = b_o.reshape(1, C)

    q_pix, kp, vp, qm, km = pl.pallas_call(
        _k1_body,
        grid=(D, N_WIN),
        in_specs=[
            pl.BlockSpec((1, HW, C, W), lambda t, wj: (t, wj, 0, 0)),
            pl.BlockSpec((C, QK_DIM), lambda t, wj: (0, 0)),
            pl.BlockSpec((C, QK_DIM), lambda t, wj: (0, 0)),
            pl.BlockSpec((C, DIM), lambda t, wj: (0, 0)),
            pl.BlockSpec((1, QK_DIM), lambda t, wj: (0, 0)),
            pl.BlockSpec((1, QK_DIM), lambda t, wj: (0, 0)),
            pl.BlockSpec((1, DIM), lambda t, wj: (0, 0)),
        ],
        out_specs=[
            pl.BlockSpec((1, N_WIN, PIX, C), lambda t, wj: (t, wj, 0, 0)),
            pl.BlockSpec((1, N_WIN, W2D, QK_DIM), lambda t, wj: (t, wj, 0, 0)),
            pl.BlockSpec((1, N_WIN, W2D, DIM), lambda t, wj: (t, wj, 0, 0)),
            pl.BlockSpec((1, N_WIN, 1, C), lambda t, wj: (t, wj, 0, 0)),
            pl.BlockSpec((1, N_WIN, 1, C), lambda t, wj: (t, wj, 0, 0)),
        ],
        out_shape=[
            jax.ShapeDtypeStruct((D, P2, PIX, C), jnp.float32),
            jax.ShapeDtypeStruct((D, P2, W2D, QK_DIM), jnp.float32),
            jax.ShapeDtypeStruct((D, P2, W2D, DIM), jnp.float32),
            jax.ShapeDtypeStruct((D, P2, 1, C), jnp.float32),
            jax.ShapeDtypeStruct((D, P2, 1, C), jnp.float32),
        ],
    )(xt, wq, wk, wv, bq, bk, bv)

    r_idx = pl.pallas_call(
        _k2_body,
        grid=(D,),
        in_specs=[
            pl.BlockSpec((1, P2, 1, C), lambda t: (t, 0, 0, 0)),
            pl.BlockSpec((1, P2, 1, C),
                         lambda t: (jnp.minimum(t + 1, D - 1), 0, 0, 0)),
        ],
        out_specs=pl.BlockSpec((1, P2, TOPK), lambda t: (t, 0, 0)),
        out_shape=jax.ShapeDtypeStruct((D, P2, TOPK), jnp.int32),
    )(qm, km)

    grid_spec = pltpu.PrefetchScalarGridSpec(
        num_scalar_prefetch=1,
        grid=(D, P2),
        in_specs=[
            pl.BlockSpec((1, 1, PIX, C), lambda t, w, s: (t, w, 0, 0)),
            pl.BlockSpec((1, P2, W2D, QK_DIM),
                         lambda t, w, s: (jnp.minimum(t + 1, D - 1), 0, 0, 0)),
            pl.BlockSpec((1, P2, W2D, DIM),
                         lambda t, w, s: (jnp.minimum(t + 1, D - 1), 0, 0, 0)),
            pl.BlockSpec((C, C), lambda t, w, s: (0, 0)),
            pl.BlockSpec((1, C), lambda t, w, s: (0, 0)),
        ],
        out_specs=pl.BlockSpec((1, HW, WW, C),
                               lambda t, w, s: (t, w // N_WIN, w % N_WIN, 0)),
    )
    out = pl.pallas_call(
        _k3_body,
        grid_spec=grid_spec,
        out_shape=jax.ShapeDtypeStruct((D, H, W, C), jnp.float32),
    )(r_idx, q_pix, kp, vp, w_o, bo)

    return out.reshape(N, D, H, W, C)


# speedup vs baseline: 4.0057x; 1.1478x over previous
import functools

import jax
import jax.numpy as jnp
from jax.experimental import pallas as pl
from jax.experimental.pallas import tpu as pltpu

N, D, H, W, C = 1, 4, 224, 224, 96
N_WIN = 14
NUM_HEADS = 8
TOPK = 4
QK_DIM = C
DIM = C
SCALE = QK_DIM ** (-0.5)
R = 4

HW = H // N_WIN
WW = W // N_WIN
P2 = N_WIN * N_WIN
PIX = HW * WW
HD, WD = HW // R, WW // R
W2D = HD * WD
CPH = QK_DIM // NUM_HEADS
KW2 = TOPK * W2D
BD = NUM_HEADS * KW2
K3_BATCH = 2


def _dot(a, b, dims=(((1,), (0,)), ((), ())), prec=jax.lax.Precision.HIGHEST):
    return jax.lax.dot_general(a, b, dims, precision=prec,
                               preferred_element_type=jnp.float32)


_DEF = jax.lax.Precision.DEFAULT


def _k1_body(x_ref, wq_ref, wk_ref, wv_ref, bq_ref, bk_ref, bv_ref,
             q_ref, kp_ref, vp_ref, qm_ref, km_ref):
    xb = x_ref[...]
    qs = []
    ks = []
    vs = []
    for r in range(HW):
        x_r = xb[0, r]
        qs.append(_dot(x_r, wq_ref[...], (((0,), (0,)), ((), ())), prec=_DEF)
                  + bq_ref[...])
        ks.append(_dot(x_r, wk_ref[...], (((0,), (0,)), ((), ())), prec=_DEF)
                  + bk_ref[...])
        vs.append(_dot(x_r, wv_ref[...], (((0,), (0,)), ((), ())), prec=_DEF)
                  + bv_ref[...])
    Q = jnp.stack(qs, axis=0)
    qw = Q.reshape(HW, N_WIN, WW, C).transpose(1, 0, 2, 3).reshape(1, N_WIN, PIX, C)
    q_ref[...] = qw
    pc_op = jnp.where(
        jax.lax.broadcasted_iota(jnp.int32, (W // R, W), 0)
        == jax.lax.broadcasted_iota(jnp.int32, (W // R, W), 1) // R, 1.0, 0.0)
    for dst, srcs in ((kp_ref, ks), (vp_ref, vs)):
        pooled = []
        for pr in range(HD):
            rsumT = srcs[R * pr]
            for dr in range(1, R):
                rsumT = rsumT + srcs[R * pr + dr]
            pooled.append(_dot(pc_op, rsumT))
        pool = jnp.stack(pooled, axis=0) * (1.0 / (R * R))
        pool = pool.reshape(HD, N_WIN, WD, C).transpose(1, 0, 2, 3)
        dst[...] = pool.reshape(1, N_WIN, W2D, C)
    m_op = jnp.where(
        jax.lax.broadcasted_iota(jnp.int32, (N_WIN, W), 0)
        == jax.lax.broadcasted_iota(jnp.int32, (N_WIN, W), 1) // WW, 1.0, 0.0)
    sq = qs[0]
    sk = ks[0]
    for r in range(1, HW):
        sq = sq + qs[r]
        sk = sk + ks[r]
    qm_ref[...] = (_dot(m_op, sq) * (1.0 / PIX)).reshape(1, N_WIN, 1, C)
    km_ref[...] = (_dot(m_op, sk) * (1.0 / PIX)).reshape(1, N_WIN, 1, C)


def _k2_body(qm_ref, km_ref, idx_ref):
    qw = qm_ref[...].reshape(P2, C)
    kw = km_ref[...].reshape(P2, C)
    lg = _dot(qw * SCALE, kw, (((1,), (1,)), ((), ())), prec=_DEF)
    cols = jax.lax.broadcasted_iota(jnp.int32, (P2, P2), 1)
    cur = lg
    picks = []
    for _ in range(TOPK):
        m = jnp.max(cur, axis=1, keepdims=True)
        eq = cur >= m
        am = jnp.min(jnp.where(eq, cols, jnp.int32(2 ** 30)), axis=1, keepdims=True)
        picks.append(am)
        cur = jnp.where(cols == am, -jnp.inf, cur)
    idx_ref[...] = jnp.concatenate(picks, axis=1).reshape(1, P2, TOPK)


def _attn_one(q, ks, vs, wo, bo):
    rows = jax.lax.broadcasted_iota(jnp.int32, (BD, C), 0) // KW2
    colh = jax.lax.broadcasted_iota(jnp.int32, (BD, C), 1) // CPH
    hm = rows == colh
    Kb = jnp.where(hm, jnp.concatenate([ks] * NUM_HEADS, axis=0), 0.0)
    L = _dot(q * SCALE, Kb, (((1,), (1,)), ((), ())), prec=_DEF)
    m = jnp.max(L, axis=1, keepdims=True)
    e = jnp.exp(L - m)
    eb = jnp.where(
        jax.lax.broadcasted_iota(jnp.int32, (BD, NUM_HEADS), 0) // KW2
        == jax.lax.broadcasted_iota(jnp.int32, (BD, NUM_HEADS), 1), 1.0, 0.0)
    s = _dot(e, eb, prec=_DEF)
    rec = 1.0 / jnp.maximum(s, 1e-30)
    Vb = jnp.where(hm, jnp.concatenate([vs] * NUM_HEADS, axis=0), 0.0)
    ou = _dot(e, Vb, prec=_DEF)
    ex = jnp.where(
        jax.lax.broadcasted_iota(jnp.int32, (NUM_HEADS, C), 0)
        == jax.lax.broadcasted_iota(jnp.int32, (NUM_HEADS, C), 1) // CPH, 1.0, 0.0)
    out = ou * _dot(rec, ex, prec=_DEF)
    o = _dot(out, wo, prec=_DEF) + bo
    return o.reshape(HW, WW, C)


def _k3_body(sref, q_ref, kp_ref, vp_ref, wo_ref, bo_ref, o_ref):
    t = pl.program_id(0)
    w2 = pl.program_id(1)
    wo = wo_ref[...]
    bo = bo_ref[...]
    outs = []
    for j in range(K3_BATCH):
        w = w2 * K3_BATCH + j
        q = q_ref[...][0, j]
        krows = []
        vrows = []
        for kk in range(TOPK):
            i = sref[t, w, kk]
            krows.append(kp_ref[0, i])
            vrows.append(vp_ref[0, i])
        ks = jnp.concatenate(krows, axis=0)
        vs = jnp.concatenate(vrows, axis=0)
        outs.append(_attn_one(q, ks, vs, wo, bo))
    o_ref[...] = jnp.concatenate(outs, axis=1).reshape(1, HW, K3_BATCH * WW, C)


@jax.jit
def kernel(x, w_qkv, b_qkv, w_o, b_o):
    xt = jnp.transpose(x.reshape(D, H, W, C), (0, 1, 3, 2))
    wq = w_qkv[:, :QK_DIM]
    wk = w_qkv[:, QK_DIM:2 * QK_DIM]
    wv = w_qkv[:, 2 * QK_DIM:]
    bq = b_qkv[:QK_DIM].reshape(1, QK_DIM)
    bk = b_qkv[QK_DIM:2 * QK_DIM].reshape(1, QK_DIM)
    bv = b_qkv[2 * QK_DIM:].reshape(1, DIM)
    bo = b_o.reshape(1, C)

    q_pix, kp, vp, qm, km = pl.pallas_call(
        _k1_body,
        grid=(D, N_WIN),
        in_specs=[
            pl.BlockSpec((1, HW, C, W), lambda t, wj: (t, wj, 0, 0)),
            pl.BlockSpec((C, QK_DIM), lambda t, wj: (0, 0)),
            pl.BlockSpec((C, QK_DIM), lambda t, wj: (0, 0)),
            pl.BlockSpec((C, DIM), lambda t, wj: (0, 0)),
            pl.BlockSpec((1, QK_DIM), lambda t, wj: (0, 0)),
            pl.BlockSpec((1, QK_DIM), lambda t, wj: (0, 0)),
            pl.BlockSpec((1, DIM), lambda t, wj: (0, 0)),
        ],
        out_specs=[
            pl.BlockSpec((1, N_WIN, PIX, C), lambda t, wj: (t, wj, 0, 0)),
            pl.BlockSpec((1, N_WIN, W2D, QK_DIM), lambda t, wj: (t, wj, 0, 0)),
            pl.BlockSpec((1, N_WIN, W2D, DIM), lambda t, wj: (t, wj, 0, 0)),
            pl.BlockSpec((1, N_WIN, 1, C), lambda t, wj: (t, wj, 0, 0)),
            pl.BlockSpec((1, N_WIN, 1, C), lambda t, wj: (t, wj, 0, 0)),
        ],
        out_shape=[
            jax.ShapeDtypeStruct((D, P2, PIX, C), jnp.float32),
            jax.ShapeDtypeStruct((D, P2, W2D, QK_DIM), jnp.float32),
            jax.ShapeDtypeStruct((D, P2, W2D, DIM), jnp.float32),
            jax.ShapeDtypeStruct((D, P2, 1, C), jnp.float32),
            jax.ShapeDtypeStruct((D, P2, 1, C), jnp.float32),
        ],
    )(xt, wq, wk, wv, bq, bk, bv)

    r_idx = pl.pallas_call(
        _k2_body,
        grid=(D,),
        in_specs=[
            pl.BlockSpec((1, P2, 1, C), lambda t: (t, 0, 0, 0)),
            pl.BlockSpec((1, P2, 1, C),
                         lambda t: (jnp.minimum(t + 1, D - 1), 0, 0, 0)),
        ],
        out_specs=pl.BlockSpec((1, P2, TOPK), lambda t: (t, 0, 0)),
        out_shape=jax.ShapeDtypeStruct((D, P2, TOPK), jnp.int32),
    )(qm, km)

    npair = N_WIN // K3_BATCH
    grid_spec = pltpu.PrefetchScalarGridSpec(
        num_scalar_prefetch=1,
        grid=(D, P2 // K3_BATCH),
        in_specs=[
            pl.BlockSpec((1, K3_BATCH, PIX, C), lambda t, w2, s: (t, w2, 0, 0)),
            pl.BlockSpec((1, P2, W2D, QK_DIM),
                         lambda t, w2, s: (jnp.minimum(t + 1, D - 1), 0, 0, 0)),
            pl.BlockSpec((1, P2, W2D, DIM),
                         lambda t, w2, s: (jnp.minimum(t + 1, D - 1), 0, 0, 0)),
            pl.BlockSpec((C, C), lambda t, w2, s: (0, 0)),
            pl.BlockSpec((1, C), lambda t, w2, s: (0, 0)),
        ],
        out_specs=pl.BlockSpec((1, HW, K3_BATCH * WW, C),
                               lambda t, w2, s: (t, w2 // npair, w2 % npair, 0)),
    )
    out = pl.pallas_call(
        _k3_body,
        grid_spec=grid_spec,
        out_shape=jax.ShapeDtypeStruct((D, H, W, C), jnp.float32),
    )(r_idx, q_pix, kp, vp, w_o, bo)

    return out.reshape(N, D, H, W, C)


# speedup vs baseline: 4.4278x; 1.1054x over previous
import functools

import jax
import jax.numpy as jnp
from jax.experimental import pallas as pl
from jax.experimental.pallas import tpu as pltpu

N, D, H, W, C = 1, 4, 224, 224, 96
N_WIN = 14
NUM_HEADS = 8
TOPK = 4
QK_DIM = C
DIM = C
SCALE = QK_DIM ** (-0.5)
R = 4

HW = H // N_WIN
WW = W // N_WIN
P2 = N_WIN * N_WIN
PIX = HW * WW
HD, WD = HW // R, WW // R
W2D = HD * WD
CPH = QK_DIM // NUM_HEADS
KW2 = TOPK * W2D
BD = NUM_HEADS * KW2
K3_BATCH = 7


def _dot(a, b, dims=(((1,), (0,)), ((), ())), prec=jax.lax.Precision.HIGHEST):
    return jax.lax.dot_general(a, b, dims, precision=prec,
                               preferred_element_type=jnp.float32)


_DEF = jax.lax.Precision.DEFAULT


def _k1_body(x_ref, wq_ref, wk_ref, wv_ref, bq_ref, bk_ref, bv_ref,
             q_ref, kp_ref, vp_ref, qm_ref, km_ref):
    xb = x_ref[...]
    qs = []
    ks = []
    vs = []
    for r in range(HW):
        x_r = xb[0, r]
        qs.append(_dot(x_r, wq_ref[...], (((0,), (0,)), ((), ())), prec=_DEF)
                  + bq_ref[...])
        ks.append(_dot(x_r, wk_ref[...], (((0,), (0,)), ((), ())), prec=_DEF)
                  + bk_ref[...])
        vs.append(_dot(x_r, wv_ref[...], (((0,), (0,)), ((), ())), prec=_DEF)
                  + bv_ref[...])
    Q = jnp.stack(qs, axis=0)
    qw = Q.reshape(HW, N_WIN, WW, C).transpose(1, 0, 2, 3).reshape(1, N_WIN, PIX, C)
    q_ref[...] = qw
    pc_op = jnp.where(
        jax.lax.broadcasted_iota(jnp.int32, (W // R, W), 0)
        == jax.lax.broadcasted_iota(jnp.int32, (W // R, W), 1) // R, 1.0, 0.0)
    for dst, srcs in ((kp_ref, ks), (vp_ref, vs)):
        pooled = []
        for pr in range(HD):
            rsumT = srcs[R * pr]
            for dr in range(1, R):
                rsumT = rsumT + srcs[R * pr + dr]
            pooled.append(_dot(pc_op, rsumT))
        pool = jnp.stack(pooled, axis=0) * (1.0 / (R * R))
        pool = pool.reshape(HD, N_WIN, WD, C).transpose(1, 0, 2, 3)
        dst[...] = pool.reshape(1, N_WIN, W2D, C)
    m_op = jnp.where(
        jax.lax.broadcasted_iota(jnp.int32, (N_WIN, W), 0)
        == jax.lax.broadcasted_iota(jnp.int32, (N_WIN, W), 1) // WW, 1.0, 0.0)
    sq = qs[0]
    sk = ks[0]
    for r in range(1, HW):
        sq = sq + qs[r]
        sk = sk + ks[r]
    qm_ref[...] = (_dot(m_op, sq) * (1.0 / PIX)).reshape(1, N_WIN, 1, C)
    km_ref[...] = (_dot(m_op, sk) * (1.0 / PIX)).reshape(1, N_WIN, 1, C)


def _k2_body(qm_ref, km_ref, idx_ref):
    qw = qm_ref[...].reshape(P2, C)
    kw = km_ref[...].reshape(P2, C)
    lg = _dot(qw * SCALE, kw, (((1,), (1,)), ((), ())), prec=_DEF)
    cols = jax.lax.broadcasted_iota(jnp.int32, (P2, P2), 1)
    cur = lg
    picks = []
    for _ in range(TOPK):
        m = jnp.max(cur, axis=1, keepdims=True)
        eq = cur >= m
        am = jnp.min(jnp.where(eq, cols, jnp.int32(2 ** 30)), axis=1, keepdims=True)
        picks.append(am)
        cur = jnp.where(cols == am, -jnp.inf, cur)
    idx_ref[...] = jnp.concatenate(picks, axis=1).reshape(1, P2, TOPK)


def _attn_one(q, ks, vs, wo, bo):
    rows = jax.lax.broadcasted_iota(jnp.int32, (BD, C), 0) // KW2
    colh = jax.lax.broadcasted_iota(jnp.int32, (BD, C), 1) // CPH
    hm = rows == colh
    Kb = jnp.where(hm, jnp.concatenate([ks] * NUM_HEADS, axis=0), 0.0)
    L = _dot(q * SCALE, Kb, (((1,), (1,)), ((), ())), prec=_DEF)
    m = jnp.max(L, axis=1, keepdims=True)
    e = jnp.exp(L - m)
    eb = jnp.where(
        jax.lax.broadcasted_iota(jnp.int32, (BD, NUM_HEADS), 0) // KW2
        == jax.lax.broadcasted_iota(jnp.int32, (BD, NUM_HEADS), 1), 1.0, 0.0)
    s = _dot(e, eb, prec=_DEF)
    rec = 1.0 / jnp.maximum(s, 1e-30)
    Vb = jnp.where(hm, jnp.concatenate([vs] * NUM_HEADS, axis=0), 0.0)
    ou = _dot(e, Vb, prec=_DEF)
    ex = jnp.where(
        jax.lax.broadcasted_iota(jnp.int32, (NUM_HEADS, C), 0)
        == jax.lax.broadcasted_iota(jnp.int32, (NUM_HEADS, C), 1) // CPH, 1.0, 0.0)
    out = ou * _dot(rec, ex, prec=_DEF)
    o = _dot(out, wo, prec=_DEF) + bo
    return o.reshape(HW, WW, C)


def _k3_body(sref, q_ref, kp_ref, vp_ref, wo_ref, bo_ref, o_ref):
    t = pl.program_id(0)
    w2 = pl.program_id(1)
    wo = wo_ref[...]
    bo = bo_ref[...]
    outs = []
    for j in range(K3_BATCH):
        w = w2 * K3_BATCH + j
        q = q_ref[...][0, j]
        krows = []
        vrows = []
        for kk in range(TOPK):
            i = sref[t, w, kk]
            krows.append(kp_ref[0, i])
            vrows.append(vp_ref[0, i])
        ks = jnp.concatenate(krows, axis=0)
        vs = jnp.concatenate(vrows, axis=0)
        outs.append(_attn_one(q, ks, vs, wo, bo))
    o_ref[...] = jnp.concatenate(outs, axis=1).reshape(1, HW, K3_BATCH * WW, C)


@jax.jit
def kernel(x, w_qkv, b_qkv, w_o, b_o):
    xt = jnp.transpose(x.reshape(D, H, W, C), (0, 1, 3, 2))
    wq = w_qkv[:, :QK_DIM]
    wk = w_qkv[:, QK_DIM:2 * QK_DIM]
    wv = w_qkv[:, 2 * QK_DIM:]
    bq = b_qkv[:QK_DIM].reshape(1, QK_DIM)
    bk = b_qkv[QK_DIM:2 * QK_DIM].reshape(1, QK_DIM)
    bv = b_qkv[2 * QK_DIM:].reshape(1, DIM)
    bo = b_o.reshape(1, C)

    q_pix, kp, vp, qm, km = pl.pallas_call(
        _k1_body,
        grid=(D, N_WIN),
        in_specs=[
            pl.BlockSpec((1, HW, C, W), lambda t, wj: (t, wj, 0, 0)),
            pl.BlockSpec((C, QK_DIM), lambda t, wj: (0, 0)),
            pl.BlockSpec((C, QK_DIM), lambda t, wj: (0, 0)),
            pl.BlockSpec((C, DIM), lambda t, wj: (0, 0)),
            pl.BlockSpec((1, QK_DIM), lambda t, wj: (0, 0)),
            pl.BlockSpec((1, QK_DIM), lambda t, wj: (0, 0)),
            pl.BlockSpec((1, DIM), lambda t, wj: (0, 0)),
        ],
        out_specs=[
            pl.BlockSpec((1, N_WIN, PIX, C), lambda t, wj: (t, wj, 0, 0)),
            pl.BlockSpec((1, N_WIN, W2D, QK_DIM), lambda t, wj: (t, wj, 0, 0)),
            pl.BlockSpec((1, N_WIN, W2D, DIM), lambda t, wj: (t, wj, 0, 0)),
            pl.BlockSpec((1, N_WIN, 1, C), lambda t, wj: (t, wj, 0, 0)),
            pl.BlockSpec((1, N_WIN, 1, C), lambda t, wj: (t, wj, 0, 0)),
        ],
        out_shape=[
            jax.ShapeDtypeStruct((D, P2, PIX, C), jnp.float32),
            jax.ShapeDtypeStruct((D, P2, W2D, QK_DIM), jnp.float32),
            jax.ShapeDtypeStruct((D, P2, W2D, DIM), jnp.float32),
            jax.ShapeDtypeStruct((D, P2, 1, C), jnp.float32),
            jax.ShapeDtypeStruct((D, P2, 1, C), jnp.float32),
        ],
    )(xt, wq, wk, wv, bq, bk, bv)

    r_idx = pl.pallas_call(
        _k2_body,
        grid=(D,),
        in_specs=[
            pl.BlockSpec((1, P2, 1, C), lambda t: (t, 0, 0, 0)),
            pl.BlockSpec((1, P2, 1, C),
                         lambda t: (jnp.minimum(t + 1, D - 1), 0, 0, 0)),
        ],
        out_specs=pl.BlockSpec((1, P2, TOPK), lambda t: (t, 0, 0)),
        out_shape=jax.ShapeDtypeStruct((D, P2, TOPK), jnp.int32),
    )(qm, km)

    npair = N_WIN // K3_BATCH
    grid_spec = pltpu.PrefetchScalarGridSpec(
        num_scalar_prefetch=1,
        grid=(D, P2 // K3_BATCH),
        in_specs=[
            pl.BlockSpec((1, K3_BATCH, PIX, C), lambda t, w2, s: (t, w2, 0, 0)),
            pl.BlockSpec((1, P2, W2D, QK_DIM),
                         lambda t, w2, s: (jnp.minimum(t + 1, D - 1), 0, 0, 0)),
            pl.BlockSpec((1, P2, W2D, DIM),
                         lambda t, w2, s: (jnp.minimum(t + 1, D - 1), 0, 0, 0)),
            pl.BlockSpec((C, C), lambda t, w2, s: (0, 0)),
            pl.BlockSpec((1, C), lambda t, w2, s: (0, 0)),
        ],
        out_specs=pl.BlockSpec((1, HW, K3_BATCH * WW, C),
                               lambda t, w2, s: (t, w2 // npair, w2 % npair, 0)),
    )
    out = pl.pallas_call(
        _k3_body,
        grid_spec=grid_spec,
        out_shape=jax.ShapeDtypeStruct((D, H, W, C), jnp.float32),
    )(r_idx, q_pix, kp, vp, w_o, bo)

    return out.reshape(N, D, H, W, C)


# speedup vs baseline: 4.5477x; 1.0271x over previous
import functools

import jax
import jax.numpy as jnp
from jax.experimental import pallas as pl
from jax.experimental.pallas import tpu as pltpu

N, D, H, W, C = 1, 4, 224, 224, 96
N_WIN = 14
NUM_HEADS = 8
TOPK = 4
QK_DIM = C
DIM = C
SCALE = QK_DIM ** (-0.5)
R = 4

HW = H // N_WIN
WW = W // N_WIN
P2 = N_WIN * N_WIN
PIX = HW * WW
HD, WD = HW // R, WW // R
W2D = HD * WD
CPH = QK_DIM // NUM_HEADS
KW2 = TOPK * W2D
BD = NUM_HEADS * KW2
K3_BATCH = 14


def _dot(a, b, dims=(((1,), (0,)), ((), ())), prec=jax.lax.Precision.HIGHEST):
    return jax.lax.dot_general(a, b, dims, precision=prec,
                               preferred_element_type=jnp.float32)


_DEF = jax.lax.Precision.DEFAULT


def _k1_body(x_ref, wq_ref, wk_ref, wv_ref, bq_ref, bk_ref, bv_ref,
             q_ref, kp_ref, vp_ref, qm_ref, km_ref):
    xb = x_ref[...]
    qs = []
    ks = []
    vs = []
    for r in range(HW):
        x_r = xb[0, r]
        qs.append(_dot(x_r, wq_ref[...], (((0,), (0,)), ((), ())), prec=_DEF)
                  + bq_ref[...])
        ks.append(_dot(x_r, wk_ref[...], (((0,), (0,)), ((), ())), prec=_DEF)
                  + bk_ref[...])
        vs.append(_dot(x_r, wv_ref[...], (((0,), (0,)), ((), ())), prec=_DEF)
                  + bv_ref[...])
    Q = jnp.stack(qs, axis=0)
    qw = Q.reshape(HW, N_WIN, WW, C).transpose(1, 0, 2, 3).reshape(1, N_WIN, PIX, C)
    q_ref[...] = qw
    pc_op = jnp.where(
        jax.lax.broadcasted_iota(jnp.int32, (W // R, W), 0)
        == jax.lax.broadcasted_iota(jnp.int32, (W // R, W), 1) // R, 1.0, 0.0)
    for dst, srcs in ((kp_ref, ks), (vp_ref, vs)):
        pooled = []
        for pr in range(HD):
            rsumT = srcs[R * pr]
            for dr in range(1, R):
                rsumT = rsumT + srcs[R * pr + dr]
            pooled.append(_dot(pc_op, rsumT))
        pool = jnp.stack(pooled, axis=0) * (1.0 / (R * R))
        pool = pool.reshape(HD, N_WIN, WD, C).transpose(1, 0, 2, 3)
        dst[...] = pool.reshape(1, N_WIN, W2D, C)
    m_op = jnp.where(
        jax.lax.broadcasted_iota(jnp.int32, (N_WIN, W), 0)
        == jax.lax.broadcasted_iota(jnp.int32, (N_WIN, W), 1) // WW, 1.0, 0.0)
    sq = qs[0]
    sk = ks[0]
    for r in range(1, HW):
        sq = sq + qs[r]
        sk = sk + ks[r]
    qm_ref[...] = (_dot(m_op, sq) * (1.0 / PIX)).reshape(1, N_WIN, 1, C)
    km_ref[...] = (_dot(m_op, sk) * (1.0 / PIX)).reshape(1, N_WIN, 1, C)


def _k2_body(qm_ref, km_ref, idx_ref):
    qw = qm_ref[...].reshape(P2, C)
    kw = km_ref[...].reshape(P2, C)
    lg = _dot(qw * SCALE, kw, (((1,), (1,)), ((), ())), prec=_DEF)
    cols = jax.lax.broadcasted_iota(jnp.int32, (P2, P2), 1)
    cur = lg
    picks = []
    for _ in range(TOPK):
        m = jnp.max(cur, axis=1, keepdims=True)
        eq = cur >= m
        am = jnp.min(jnp.where(eq, cols, jnp.int32(2 ** 30)), axis=1, keepdims=True)
        picks.append(am)
        cur = jnp.where(cols == am, -jnp.inf, cur)
    idx_ref[...] = jnp.concatenate(picks, axis=1).reshape(1, P2, TOPK)


def _attn_one(q, ks, vs, wo, bo):
    rows = jax.lax.broadcasted_iota(jnp.int32, (BD, C), 0) // KW2
    colh = jax.lax.broadcasted_iota(jnp.int32, (BD, C), 1) // CPH
    hm = rows == colh
    Kb = jnp.where(hm, jnp.concatenate([ks] * NUM_HEADS, axis=0), 0.0)
    L = _dot(q * SCALE, Kb, (((1,), (1,)), ((), ())), prec=_DEF)
    m = jnp.max(L, axis=1, keepdims=True)
    e = jnp.exp(L - m)
    eb = jnp.where(
        jax.lax.broadcasted_iota(jnp.int32, (BD, NUM_HEADS), 0) // KW2
        == jax.lax.broadcasted_iota(jnp.int32, (BD, NUM_HEADS), 1), 1.0, 0.0)
    s = _dot(e, eb, prec=_DEF)
    rec = 1.0 / jnp.maximum(s, 1e-30)
    Vb = jnp.where(hm, jnp.concatenate([vs] * NUM_HEADS, axis=0), 0.0)
    ou = _dot(e, Vb, prec=_DEF)
    ex = jnp.where(
        jax.lax.broadcasted_iota(jnp.int32, (NUM_HEADS, C), 0)
        == jax.lax.broadcasted_iota(jnp.int32, (NUM_HEADS, C), 1) // CPH, 1.0, 0.0)
    out = ou * _dot(rec, ex, prec=_DEF)
    o = _dot(out, wo, prec=_DEF) + bo
    return o.reshape(HW, WW, C)


def _k3_body(sref, q_ref, kp_ref, vp_ref, wo_ref, bo_ref, o_ref):
    t = pl.program_id(0)
    w2 = pl.program_id(1)
    wo = wo_ref[...]
    bo = bo_ref[...]
    outs = []
    for j in range(K3_BATCH):
        w = w2 * K3_BATCH + j
        q = q_ref[...][0, j]
        krows = []
        vrows = []
        for kk in range(TOPK):
            i = sref[t, w, kk]
            krows.append(kp_ref[0, i])
            vrows.append(vp_ref[0, i])
        ks = jnp.concatenate(krows, axis=0)
        vs = jnp.concatenate(vrows, axis=0)
        outs.append(_attn_one(q, ks, vs, wo, bo))
    o_ref[...] = jnp.concatenate(outs, axis=1).reshape(1, HW, K3_BATCH * WW, C)


@jax.jit
def kernel(x, w_qkv, b_qkv, w_o, b_o):
    xt = jnp.transpose(x.reshape(D, H, W, C), (0, 1, 3, 2))
    wq = w_qkv[:, :QK_DIM]
    wk = w_qkv[:, QK_DIM:2 * QK_DIM]
    wv = w_qkv[:, 2 * QK_DIM:]
    bq = b_qkv[:QK_DIM].reshape(1, QK_DIM)
    bk = b_qkv[QK_DIM:2 * QK_DIM].reshape(1, QK_DIM)
    bv = b_qkv[2 * QK_DIM:].reshape(1, DIM)
    bo = b_o.reshape(1, C)

    q_pix, kp, vp, qm, km = pl.pallas_call(
        _k1_body,
        grid=(D, N_WIN),
        in_specs=[
            pl.BlockSpec((1, HW, C, W), lambda t, wj: (t, wj, 0, 0)),
            pl.BlockSpec((C, QK_DIM), lambda t, wj: (0, 0)),
            pl.BlockSpec((C, QK_DIM), lambda t, wj: (0, 0)),
            pl.BlockSpec((C, DIM), lambda t, wj: (0, 0)),
            pl.BlockSpec((1, QK_DIM), lambda t, wj: (0, 0)),
            pl.BlockSpec((1, QK_DIM), lambda t, wj: (0, 0)),
            pl.BlockSpec((1, DIM), lambda t, wj: (0, 0)),
        ],
        out_specs=[
            pl.BlockSpec((1, N_WIN, PIX, C), lambda t, wj: (t, wj, 0, 0)),
            pl.BlockSpec((1, N_WIN, W2D, QK_DIM), lambda t, wj: (t, wj, 0, 0)),
            pl.BlockSpec((1, N_WIN, W2D, DIM), lambda t, wj: (t, wj, 0, 0)),
            pl.BlockSpec((1, N_WIN, 1, C), lambda t, wj: (t, wj, 0, 0)),
            pl.BlockSpec((1, N_WIN, 1, C), lambda t, wj: (t, wj, 0, 0)),
        ],
        out_shape=[
            jax.ShapeDtypeStruct((D, P2, PIX, C), jnp.float32),
            jax.ShapeDtypeStruct((D, P2, W2D, QK_DIM), jnp.float32),
            jax.ShapeDtypeStruct((D, P2, W2D, DIM), jnp.float32),
            jax.ShapeDtypeStruct((D, P2, 1, C), jnp.float32),
            jax.ShapeDtypeStruct((D, P2, 1, C), jnp.float32),
        ],
    )(xt, wq, wk, wv, bq, bk, bv)

    r_idx = pl.pallas_call(
        _k2_body,
        grid=(D,),
        in_specs=[
            pl.BlockSpec((1, P2, 1, C), lambda t: (t, 0, 0, 0)),
            pl.BlockSpec((1, P2, 1, C),
                         lambda t: (jnp.minimum(t + 1, D - 1), 0, 0, 0)),
        ],
        out_specs=pl.BlockSpec((1, P2, TOPK), lambda t: (t, 0, 0)),
        out_shape=jax.ShapeDtypeStruct((D, P2, TOPK), jnp.int32),
    )(qm, km)

    npair = N_WIN // K3_BATCH
    grid_spec = pltpu.PrefetchScalarGridSpec(
        num_scalar_prefetch=1,
        grid=(D, P2 // K3_BATCH),
        in_specs=[
            pl.BlockSpec((1, K3_BATCH, PIX, C), lambda t, w2, s: (t, w2, 0, 0)),
            pl.BlockSpec((1, P2, W2D, QK_DIM),
                         lambda t, w2, s: (jnp.minimum(t + 1, D - 1), 0, 0, 0)),
            pl.BlockSpec((1, P2, W2D, DIM),
                         lambda t, w2, s: (jnp.minimum(t + 1, D - 1), 0, 0, 0)),
            pl.BlockSpec((C, C), lambda t, w2, s: (0, 0)),
            pl.BlockSpec((1, C), lambda t, w2, s: (0, 0)),
        ],
        out_specs=pl.BlockSpec((1, HW, K3_BATCH * WW, C),
                               lambda t, w2, s: (t, w2 // npair, w2 % npair, 0)),
    )
    out = pl.pallas_call(
        _k3_body,
        grid_spec=grid_spec,
        out_shape=jax.ShapeDtypeStruct((D, H, W, C), jnp.float32),
    )(r_idx, q_pix, kp, vp, w_o, bo)

    return out.reshape(N, D, H, W, C)
